# baseline (speedup 1.0000x reference)
import functools

import jax
import jax.numpy as jnp
from jax import lax
from jax.experimental import pallas as pl
from jax.experimental.pallas import tpu as pltpu
from jax.experimental.pallas import tpu_sc as plsc

_N = 10000
_E = 320000
_DIN = 128
_DHID = 16
_NCLS = 10

_NPAD = 10240
_RB = 1024
_GRID = _NPAD // _RB

_NCORES = 2
_NSUB = 16
_NW = _NCORES * _NSUB
_EPW = _E // _NW
_CH = 80
_NCH = _EPW // _CH
_SCH = 128
_SNCH = _EPW // _SCH
_STAIL = _EPW - _SNCH * _SCH
_RPT = _NPAD // _NSUB
_DEGW = _DHID
_PK = 128 // _DHID
_PROWS = _NPAD // _PK
_RBP = _RB // _PK


def _mesh():
    return plsc.VectorSubcoreMesh(
        core_axis_name="c", subcore_axis_name="s",
        num_cores=_NCORES, num_subcores=_NSUB)


def _sc_degree(ei, ones, zeros16):
    @functools.partial(
        pl.kernel, mesh=_mesh(),
        compiler_params=pltpu.CompilerParams(use_tc_tiling_on_sc=False),
        out_type=jax.ShapeDtypeStruct((_NCORES, _NPAD, _DEGW), jnp.float32),
        scratch_types=[
            pltpu.VMEM((_EPW,), jnp.int32),
            pltpu.VMEM((_SCH, _DEGW), jnp.float32),
            pltpu.VMEM_SHARED((_NPAD, _DEGW), jnp.float32),
            [pltpu.SemaphoreType.DMA] * 4,
        ])
    def deg_kernel(ei_hbm, ones_hbm, z_hbm, out_hbm, didx, ones_v, acc, sem):
        cid = lax.axis_index("c")
        sid = lax.axis_index("s")
        wid = sid * _NCORES + cid
        pltpu.sync_copy(z_hbm, acc.at[pl.ds(sid * _RPT, _RPT)])
        pltpu.sync_copy(ones_hbm, ones_v)
        pltpu.sync_copy(ei_hbm.at[1, pl.ds(wid * _EPW, _EPW)], didx)
        plsc.subcore_barrier()

        def dst_at(c):
            return acc.at[didx.at[pl.ds(c * _SCH, _SCH)]]

        for k in range(4):
            pltpu.async_copy(ones_v, dst_at(k), sem[k], add=True)

        def chunk(j, c):
            for k in range(4):
                pltpu.make_async_copy(ones_v, dst_at(4 * (j - 1) + k),
                                      sem[k]).wait()
                pltpu.async_copy(ones_v, dst_at(4 * j + k), sem[k],
                                 add=True)
            return c

        lax.fori_loop(1, _SNCH // 4, chunk, None)
        c0 = 4 * (_SNCH // 4 - 1)
        tb = _SNCH * _SCH
        tail = pltpu.make_async_copy(
            ones_v.at[pl.ds(0, _STAIL)],
            acc.at[didx.at[pl.ds(tb, _STAIL)]], sem[2])
        for k in range(4):
            pltpu.make_async_copy(ones_v, dst_at(c0 + k), sem[k]).wait()
        for k in range(_SNCH - c0 - 4):
            pltpu.async_copy(ones_v, dst_at(c0 + 4 + k), sem[k], add=True)
        pltpu.async_copy(ones_v.at[pl.ds(0, _STAIL)],
                         acc.at[didx.at[pl.ds(tb, _STAIL)]], sem[2],
                         add=True)
        for k in range(_SNCH - c0 - 4):
            pltpu.make_async_copy(ones_v, dst_at(c0 + 4 + k), sem[k]).wait()
        tail.wait()
        plsc.subcore_barrier()
        pltpu.sync_copy(acc.at[pl.ds(sid * _RPT, _RPT)],
                        out_hbm.at[cid, pl.ds(sid * _RPT, _RPT)])

    return deg_kernel(ei, ones, zeros16)


def _sc_scatter(hh, ei, zeros16):
    @functools.partial(
        pl.kernel, mesh=_mesh(),
        compiler_params=pltpu.CompilerParams(use_tc_tiling_on_sc=False),
        out_type=jax.ShapeDtypeStruct((_NCORES, _NPAD, _DHID), jnp.float32),
        scratch_types=[
            pltpu.VMEM((_EPW,), jnp.int32),
            pltpu.VMEM((_EPW,), jnp.int32),
            pltpu.VMEM((4, _SCH, _DHID), jnp.float32),
            pltpu.VMEM((_STAIL, _DHID), jnp.float32),
            pltpu.VMEM_SHARED((_NPAD, _DHID), jnp.float32),
            [pltpu.SemaphoreType.DMA] * 4,
            [pltpu.SemaphoreType.DMA] * 4,
        ])
    def scat_kernel(hh_hbm, ei_hbm, z_hbm, out_hbm,
                    sidx, didx, rows, rowt, acc, gsem, ssem):
        cid = lax.axis_index("c")
        sid = lax.axis_index("s")
        wid = sid * _NCORES + cid
        pltpu.sync_copy(z_hbm, acc.at[pl.ds(sid * _RPT, _RPT)])
        pltpu.sync_copy(ei_hbm.at[0, pl.ds(wid * _EPW, _EPW)], sidx)
        pltpu.sync_copy(ei_hbm.at[1, pl.ds(wid * _EPW, _EPW)], didx)
        plsc.subcore_barrier()

        def gat(c, k):
            return pltpu.make_async_copy(
                hh_hbm.at[sidx.at[pl.ds(c * _SCH, _SCH)]], rows.at[k],
                gsem[k])

        def sca(c, k):
            return pltpu.make_async_copy(
                rows.at[k], acc.at[didx.at[pl.ds(c * _SCH, _SCH)]], ssem[k])

        def start_gat(c, k):
            pltpu.async_copy(hh_hbm.at[sidx.at[pl.ds(c * _SCH, _SCH)]],
                             rows.at[k], gsem[k])

        def start_sca(c, k):
            pltpu.async_copy(rows.at[k],
                             acc.at[didx.at[pl.ds(c * _SCH, _SCH)]],
                             ssem[k], add=True)

        for k in range(4):
            start_gat(k, k)

        def quad(j, carry):
            c0 = 4 * j
            for k in range(4):
                gat(c0 + k, k).wait()
                start_sca(c0 + k, k)
            for k in range(4):
                sca(c0 + k, k).wait()
                start_gat(c0 + 4 + k, k)
            return carry

        nq = _SNCH // 4 - 1
        lax.fori_loop(0, nq, quad, None)
        c0 = 4 * nq
        tb = _SNCH * _SCH
        for k in range(4):
            gat(c0 + k, k).wait()
            start_sca(c0 + k, k)
        for k in range(_SNCH - c0 - 4):
            sca(c0 + k, k).wait()
            start_gat(c0 + 4 + k, k)
        pltpu.async_copy(hh_hbm.at[sidx.at[pl.ds(tb, _STAIL)]], rowt,
                         gsem[2])
        for k in range(_SNCH - c0 - 4, 4):
            sca(c0 + k, k).wait()
        for k in range(_SNCH - c0 - 4):
            gat(c0 + 4 + k, k).wait()
            start_sca(c0 + 4 + k, k)
        pltpu.make_async_copy(hh_hbm.at[sidx.at[pl.ds(tb, _STAIL)]], rowt,
                              gsem[2]).wait()
        pltpu.sync_copy(rowt, acc.at[didx.at[pl.ds(tb, _STAIL)]], add=True)
        for k in range(_SNCH - c0 - 4):
            sca(c0 + 4 + k, k).wait()
        plsc.subcore_barrier()
        pltpu.sync_copy(acc.at[pl.ds(sid * _RPT, _RPT)],
                        out_hbm.at[cid, pl.ds(sid * _RPT, _RPT)])

    return scat_kernel(hh, ei, zeros16)


def _dinv_packed(deg_ref):
    dsum = deg_ref[0] + deg_ref[1] + 1.0
    return lax.rsqrt(jnp.maximum(dsum, 1.0))


def _tc1_body(xp_ref, w1_ref, deg_ref, o_ref):
    hp = jnp.dot(xp_ref[...], w1_ref[...],
                 preferred_element_type=jnp.float32)
    o_ref[...] = hp * _dinv_packed(deg_ref)


def _tc1(xp, W1blk, degp):
    return pl.pallas_call(
        _tc1_body,
        grid=(_GRID,),
        in_specs=[
            pl.BlockSpec((_RBP, _PK * _DIN), lambda i: (i, 0)),
            pl.BlockSpec((_PK * _DIN, 128), lambda i: (0, 0)),
            pl.BlockSpec((_NCORES, _RBP, 128), lambda i: (0, i, 0)),
        ],
        out_specs=pl.BlockSpec((_RBP, 128), lambda i: (i, 0)),
        out_shape=jax.ShapeDtypeStruct((_PROWS, 128), jnp.float32),
    )(xp, W1blk, degp)


def _tc2_body(acc_ref, hh_ref, deg_ref, w2_ref, b1_ref, o_ref):
    dinv = _dinv_packed(deg_ref)
    s = acc_ref[0] + acc_ref[1] + hh_ref[...]
    h1 = jnp.maximum(s * dinv + b1_ref[...], 0.0)
    h2 = jnp.dot(h1, w2_ref[...], preferred_element_type=jnp.float32)
    o_ref[...] = h2 * dinv


def _tc2(acc1p, hhp, degp, W2blk, b1p):
    return pl.pallas_call(
        _tc2_body,
        grid=(_GRID,),
        in_specs=[
            pl.BlockSpec((_NCORES, _RBP, 128), lambda i: (0, i, 0)),
            pl.BlockSpec((_RBP, 128), lambda i: (i, 0)),
            pl.BlockSpec((_NCORES, _RBP, 128), lambda i: (0, i, 0)),
            pl.BlockSpec((128, 128), lambda i: (0, 0)),
            pl.BlockSpec((1, 128), lambda i: (0, 0)),
        ],
        out_specs=pl.BlockSpec((_RBP, 128), lambda i: (i, 0)),
        out_shape=jax.ShapeDtypeStruct((_PROWS, 128), jnp.float32),
    )(acc1p, hhp, degp, W2blk, b1p)


def _tc3_body(acc_ref, hh_ref, deg_ref, b2_ref, o_ref):
    dinv = _dinv_packed(deg_ref)
    s = acc_ref[0] + acc_ref[1] + hh_ref[...]
    zp = s * dinv + b2_ref[...]
    col = lax.broadcasted_iota(jnp.int32, (_RBP, _DHID), 1)
    mask = col < _NCLS
    mparts = []
    for a in range(_PK):
        z = zp[:, a * _DHID:(a + 1) * _DHID]
        neg = jnp.full_like(z, -3.0e38)
        m = jnp.max(jnp.where(mask, z, neg), axis=1, keepdims=True)
        mparts.append(jnp.broadcast_to(m, (_RBP, _DHID)))
    mb = jnp.concatenate(mparts, axis=1)
    lane = lax.broadcasted_iota(jnp.int32, (_RBP, 128), 1)
    maskp = lax.rem(lane, _DHID) < _NCLS
    e = jnp.where(maskp, jnp.exp(zp - mb), 0.0)
    gi = lax.broadcasted_iota(jnp.int32, (128, 128), 0) // _DHID
    gj = lax.broadcasted_iota(jnp.int32, (128, 128), 1) // _DHID
    gmat = (gi == gj).astype(jnp.float32)
    gsum = jnp.dot(e, gmat, preferred_element_type=jnp.float32,
                   precision=lax.Precision.HIGHEST)
    o_ref[...] = zp - mb - jnp.log(gsum)


def _tc3(acc2p, hh2p, degp, b2p):
    return pl.pallas_call(
        _tc3_body,
        grid=(_GRID,),
        in_specs=[
            pl.BlockSpec((_NCORES, _RBP, 128), lambda i: (0, i, 0)),
            pl.BlockSpec((_RBP, 128), lambda i: (i, 0)),
            pl.BlockSpec((_NCORES, _RBP, 128), lambda i: (0, i, 0)),
            pl.BlockSpec((1, 128), lambda i: (0, 0)),
        ],
        out_specs=pl.BlockSpec((_RBP, 128), lambda i: (i, 0)),
        out_shape=jax.ShapeDtypeStruct((_PROWS, 128), jnp.float32),
    )(acc2p, hh2p, degp, b2p)


def kernel(x, edge_index, W1, b1, W2, b2):
    ei = edge_index.astype(jnp.int32)
    W2p = jnp.pad(W2, ((0, 0), (0, _DHID - _NCLS)))
    W2blk = jnp.kron(jnp.eye(_PK, dtype=jnp.float32), W2p)
    W1blk = jnp.kron(jnp.eye(_PK, dtype=jnp.float32), W1)
    xp = jnp.pad(x.reshape(_N * _DIN), (0, (_NPAD - _N) * _DIN)).reshape(
        _PROWS, _PK * _DIN)
    b1p = jnp.tile(b1, _PK).reshape(1, 128)
    b2p = jnp.tile(jnp.pad(b2, (0, _DHID - _NCLS)), _PK).reshape(1, 128)
    ones = jnp.ones((_SCH, _DEGW), jnp.float32)
    zeros16 = jnp.zeros((_RPT, _DHID), jnp.float32)

    deg_parts = _sc_degree(ei, ones, zeros16)
    degp = deg_parts.reshape(_NCORES, _PROWS, 128)
    hhp = _tc1(xp, W1blk, degp)
    acc1 = _sc_scatter(hhp.reshape(_NPAD, _DHID), ei, zeros16)
    hh2p = _tc2(acc1.reshape(_NCORES, _PROWS, 128), hhp, degp, W2blk, b1p)
    acc2 = _sc_scatter(hh2p.reshape(_NPAD, _DHID), ei, zeros16)
    outp = _tc3(acc2.reshape(_NCORES, _PROWS, 128), hh2p, degp, b2p)
    return outp.reshape(_NPAD, _DHID)[:_N, :_NCLS]


# speedup vs baseline: 1.0966x; 1.0966x over previous
import functools

import jax
import jax.numpy as jnp
from jax import lax
from jax.experimental import pallas as pl
from jax.experimental.pallas import tpu as pltpu
from jax.experimental.pallas import tpu_sc as plsc

_N = 10000
_E = 320000
_DIN = 128
_DHID = 16
_NCLS = 10

_NPAD = 10240
_RB = 1024
_GRID = _NPAD // _RB

_NCORES = 2
_NSUB = 16
_NW = _NCORES * _NSUB
_EPW = _E // _NW
_CH = 80
_NCH = _EPW // _CH
_SCH = 256
_SNCH = _EPW // _SCH
_STAIL = _EPW - _SNCH * _SCH
_RPT = _NPAD // _NSUB
_DEGW = _DHID
_PK = 128 // _DHID
_PROWS = _NPAD // _PK
_RBP = _RB // _PK


def _mesh():
    return plsc.VectorSubcoreMesh(
        core_axis_name="c", subcore_axis_name="s",
        num_cores=_NCORES, num_subcores=_NSUB)


def _sc_degree(ei, ones, zeros16):
    @functools.partial(
        pl.kernel, mesh=_mesh(),
        compiler_params=pltpu.CompilerParams(use_tc_tiling_on_sc=False),
        out_type=jax.ShapeDtypeStruct((_NCORES, _NPAD, _DEGW), jnp.float32),
        scratch_types=[
            pltpu.VMEM((_EPW,), jnp.int32),
            pltpu.VMEM((_SCH, _DEGW), jnp.float32),
            pltpu.VMEM_SHARED((_NPAD, _DEGW), jnp.float32),
            [pltpu.SemaphoreType.DMA] * 4,
        ])
    def deg_kernel(ei_hbm, ones_hbm, z_hbm, out_hbm, didx, ones_v, acc, sem):
        cid = lax.axis_index("c")
        sid = lax.axis_index("s")
        wid = sid * _NCORES + cid
        pltpu.sync_copy(z_hbm, acc.at[pl.ds(sid * _RPT, _RPT)])
        pltpu.sync_copy(ones_hbm, ones_v)
        pltpu.sync_copy(ei_hbm.at[1, pl.ds(wid * _EPW, _EPW)], didx)
        plsc.subcore_barrier()

        def dst_at(c):
            return acc.at[didx.at[pl.ds(c * _SCH, _SCH)]]

        for k in range(4):
            pltpu.async_copy(ones_v, dst_at(k), sem[k], add=True)

        def chunk(j, c):
            for k in range(4):
                pltpu.make_async_copy(ones_v, dst_at(4 * (j - 1) + k),
                                      sem[k]).wait()
                pltpu.async_copy(ones_v, dst_at(4 * j + k), sem[k],
                                 add=True)
            return c

        lax.fori_loop(1, _SNCH // 4, chunk, None)
        c0 = 4 * (_SNCH // 4 - 1)
        tb = _SNCH * _SCH
        tail = pltpu.make_async_copy(
            ones_v.at[pl.ds(0, _STAIL)],
            acc.at[didx.at[pl.ds(tb, _STAIL)]], sem[3])
        for k in range(4):
            pltpu.make_async_copy(ones_v, dst_at(c0 + k), sem[k]).wait()
        for k in range(_SNCH - c0 - 4):
            pltpu.async_copy(ones_v, dst_at(c0 + 4 + k), sem[k], add=True)
        pltpu.async_copy(ones_v.at[pl.ds(0, _STAIL)],
                         acc.at[didx.at[pl.ds(tb, _STAIL)]], sem[3],
                         add=True)
        for k in range(_SNCH - c0 - 4):
            pltpu.make_async_copy(ones_v, dst_at(c0 + 4 + k), sem[k]).wait()
        tail.wait()
        plsc.subcore_barrier()
        pltpu.sync_copy(acc.at[pl.ds(sid * _RPT, _RPT)],
                        out_hbm.at[cid, pl.ds(sid * _RPT, _RPT)])

    return deg_kernel(ei, ones, zeros16)


def _sc_scatter(hh, ei, zeros16):
    @functools.partial(
        pl.kernel, mesh=_mesh(),
        compiler_params=pltpu.CompilerParams(use_tc_tiling_on_sc=False),
        out_type=jax.ShapeDtypeStruct((_NCORES, _NPAD, _DHID), jnp.float32),
        scratch_types=[
            pltpu.VMEM((_EPW,), jnp.int32),
            pltpu.VMEM((_EPW,), jnp.int32),
            pltpu.VMEM((4, _SCH, _DHID), jnp.float32),
            pltpu.VMEM((_STAIL, _DHID), jnp.float32),
            pltpu.VMEM_SHARED((_NPAD, _DHID), jnp.float32),
            [pltpu.SemaphoreType.DMA] * 4,
            [pltpu.SemaphoreType.DMA] * 4,
        ])
    def scat_kernel(hh_hbm, ei_hbm, z_hbm, out_hbm,
                    sidx, didx, rows, rowt, acc, gsem, ssem):
        cid = lax.axis_index("c")
        sid = lax.axis_index("s")
        wid = sid * _NCORES + cid
        pltpu.sync_copy(z_hbm, acc.at[pl.ds(sid * _RPT, _RPT)])
        pltpu.sync_copy(ei_hbm.at[0, pl.ds(wid * _EPW, _EPW)], sidx)
        pltpu.sync_copy(ei_hbm.at[1, pl.ds(wid * _EPW, _EPW)], didx)
        plsc.subcore_barrier()

        def gat(c, k):
            return pltpu.make_async_copy(
                hh_hbm.at[sidx.at[pl.ds(c * _SCH, _SCH)]], rows.at[k],
                gsem[k])

        def sca(c, k):
            return pltpu.make_async_copy(
                rows.at[k], acc.at[didx.at[pl.ds(c * _SCH, _SCH)]], ssem[k])

        def start_gat(c, k):
            pltpu.async_copy(hh_hbm.at[sidx.at[pl.ds(c * _SCH, _SCH)]],
                             rows.at[k], gsem[k])

        def start_sca(c, k):
            pltpu.async_copy(rows.at[k],
                             acc.at[didx.at[pl.ds(c * _SCH, _SCH)]],
                             ssem[k], add=True)

        for k in range(4):
            start_gat(k, k)

        def quad(j, carry):
            c0 = 4 * j
            for k in range(4):
                gat(c0 + k, k).wait()
                start_sca(c0 + k, k)
            for k in range(4):
                sca(c0 + k, k).wait()
                start_gat(c0 + 4 + k, k)
            return carry

        nq = _SNCH // 4 - 1
        lax.fori_loop(0, nq, quad, None)
        c0 = 4 * nq
        tb = _SNCH * _SCH
        for k in range(4):
            gat(c0 + k, k).wait()
            start_sca(c0 + k, k)
        for k in range(_SNCH - c0 - 4):
            sca(c0 + k, k).wait()
            start_gat(c0 + 4 + k, k)
        pltpu.async_copy(hh_hbm.at[sidx.at[pl.ds(tb, _STAIL)]], rowt,
                         gsem[3])
        for k in range(_SNCH - c0 - 4, 4):
            sca(c0 + k, k).wait()
        for k in range(_SNCH - c0 - 4):
            gat(c0 + 4 + k, k).wait()
            start_sca(c0 + 4 + k, k)
        pltpu.make_async_copy(hh_hbm.at[sidx.at[pl.ds(tb, _STAIL)]], rowt,
                              gsem[3]).wait()
        pltpu.sync_copy(rowt, acc.at[didx.at[pl.ds(tb, _STAIL)]], add=True)
        for k in range(_SNCH - c0 - 4):
            sca(c0 + 4 + k, k).wait()
        plsc.subcore_barrier()
        pltpu.sync_copy(acc.at[pl.ds(sid * _RPT, _RPT)],
                        out_hbm.at[cid, pl.ds(sid * _RPT, _RPT)])

    return scat_kernel(hh, ei, zeros16)


def _dinv_packed(deg_ref):
    dsum = deg_ref[0] + deg_ref[1] + 1.0
    return lax.rsqrt(jnp.maximum(dsum, 1.0))


def _tc1_body(xp_ref, w1_ref, deg_ref, o_ref):
    hp = jnp.dot(xp_ref[...], w1_ref[...],
                 preferred_element_type=jnp.float32)
    o_ref[...] = hp * _dinv_packed(deg_ref)


def _tc1(xp, W1blk, degp):
    return pl.pallas_call(
        _tc1_body,
        grid=(_GRID,),
        in_specs=[
            pl.BlockSpec((_RBP, _PK * _DIN), lambda i: (i, 0)),
            pl.BlockSpec((_PK * _DIN, 128), lambda i: (0, 0)),
            pl.BlockSpec((_NCORES, _RBP, 128), lambda i: (0, i, 0)),
        ],
        out_specs=pl.BlockSpec((_RBP, 128), lambda i: (i, 0)),
        out_shape=jax.ShapeDtypeStruct((_PROWS, 128), jnp.float32),
    )(xp, W1blk, degp)


def _tc2_body(acc_ref, hh_ref, deg_ref, w2_ref, b1_ref, o_ref):
    dinv = _dinv_packed(deg_ref)
    s = acc_ref[0] + acc_ref[1] + hh_ref[...]
    h1 = jnp.maximum(s * dinv + b1_ref[...], 0.0)
    h2 = jnp.dot(h1, w2_ref[...], preferred_element_type=jnp.float32)
    o_ref[...] = h2 * dinv


def _tc2(acc1p, hhp, degp, W2blk, b1p):
    return pl.pallas_call(
        _tc2_body,
        grid=(_GRID,),
        in_specs=[
            pl.BlockSpec((_NCORES, _RBP, 128), lambda i: (0, i, 0)),
            pl.BlockSpec((_RBP, 128), lambda i: (i, 0)),
            pl.BlockSpec((_NCORES, _RBP, 128), lambda i: (0, i, 0)),
            pl.BlockSpec((128, 128), lambda i: (0, 0)),
            pl.BlockSpec((1, 128), lambda i: (0, 0)),
        ],
        out_specs=pl.BlockSpec((_RBP, 128), lambda i: (i, 0)),
        out_shape=jax.ShapeDtypeStruct((_PROWS, 128), jnp.float32),
    )(acc1p, hhp, degp, W2blk, b1p)


def _tc3_body(acc_ref, hh_ref, deg_ref, b2_ref, o_ref):
    dinv = _dinv_packed(deg_ref)
    s = acc_ref[0] + acc_ref[1] + hh_ref[...]
    zp = s * dinv + b2_ref[...]
    col = lax.broadcasted_iota(jnp.int32, (_RBP, _DHID), 1)
    mask = col < _NCLS
    mparts = []
    for a in range(_PK):
        z = zp[:, a * _DHID:(a + 1) * _DHID]
        neg = jnp.full_like(z, -3.0e38)
        m = jnp.max(jnp.where(mask, z, neg), axis=1, keepdims=True)
        mparts.append(jnp.broadcast_to(m, (_RBP, _DHID)))
    mb = jnp.concatenate(mparts, axis=1)
    lane = lax.broadcasted_iota(jnp.int32, (_RBP, 128), 1)
    maskp = lax.rem(lane, _DHID) < _NCLS
    e = jnp.where(maskp, jnp.exp(zp - mb), 0.0)
    gi = lax.broadcasted_iota(jnp.int32, (128, 128), 0) // _DHID
    gj = lax.broadcasted_iota(jnp.int32, (128, 128), 1) // _DHID
    gmat = (gi == gj).astype(jnp.float32)
    gsum = jnp.dot(e, gmat, preferred_element_type=jnp.float32,
                   precision=lax.Precision.HIGHEST)
    o_ref[...] = zp - mb - jnp.log(gsum)


def _tc3(acc2p, hh2p, degp, b2p):
    return pl.pallas_call(
        _tc3_body,
        grid=(_GRID,),
        in_specs=[
            pl.BlockSpec((_NCORES, _RBP, 128), lambda i: (0, i, 0)),
            pl.BlockSpec((_RBP, 128), lambda i: (i, 0)),
            pl.BlockSpec((_NCORES, _RBP, 128), lambda i: (0, i, 0)),
            pl.BlockSpec((1, 128), lambda i: (0, 0)),
        ],
        out_specs=pl.BlockSpec((_RBP, 128), lambda i: (i, 0)),
        out_shape=jax.ShapeDtypeStruct((_PROWS, 128), jnp.float32),
    )(acc2p, hh2p, degp, b2p)


def kernel(x, edge_index, W1, b1, W2, b2):
    ei = edge_index.astype(jnp.int32)
    W2p = jnp.pad(W2, ((0, 0), (0, _DHID - _NCLS)))
    W2blk = jnp.kron(jnp.eye(_PK, dtype=jnp.float32), W2p)
    W1blk = jnp.kron(jnp.eye(_PK, dtype=jnp.float32), W1)
    xp = jnp.pad(x.reshape(_N * _DIN), (0, (_NPAD - _N) * _DIN)).reshape(
        _PROWS, _PK * _DIN)
    b1p = jnp.tile(b1, _PK).reshape(1, 128)
    b2p = jnp.tile(jnp.pad(b2, (0, _DHID - _NCLS)), _PK).reshape(1, 128)
    ones = jnp.ones((_SCH, _DEGW), jnp.float32)
    zeros16 = jnp.zeros((_RPT, _DHID), jnp.float32)

    deg_parts = _sc_degree(ei, ones, zeros16)
    degp = deg_parts.reshape(_NCORES, _PROWS, 128)
    hhp = _tc1(xp, W1blk, degp)
    acc1 = _sc_scatter(hhp.reshape(_NPAD, _DHID), ei, zeros16)
    hh2p = _tc2(acc1.reshape(_NCORES, _PROWS, 128), hhp, degp, W2blk, b1p)
    acc2 = _sc_scatter(hh2p.reshape(_NPAD, _DHID), ei, zeros16)
    outp = _tc3(acc2.reshape(_NCORES, _PROWS, 128), hh2p, degp, b2p)
    return outp.reshape(_NPAD, _DHID)[:_N, :_NCLS]


# speedup vs baseline: 1.1274x; 1.0281x over previous
import functools

import jax
import jax.numpy as jnp
from jax import lax
from jax.experimental import pallas as pl
from jax.experimental.pallas import tpu as pltpu
from jax.experimental.pallas import tpu_sc as plsc

_N = 10000
_E = 320000
_DIN = 128
_DHID = 16
_NCLS = 10

_NPAD = 10240
_RB = 1024
_GRID = _NPAD // _RB

_NCORES = 2
_NSUB = 16
_NW = _NCORES * _NSUB
_EPW = _E // _NW
_CH = 80
_NCH = _EPW // _CH
_SCH = 496
_SNCH = _EPW // _SCH
_STAIL = _EPW - _SNCH * _SCH
_RPT = _NPAD // _NSUB
_DEGW = _DHID
_PK = 128 // _DHID
_PROWS = _NPAD // _PK
_RBP = _RB // _PK


def _mesh():
    return plsc.VectorSubcoreMesh(
        core_axis_name="c", subcore_axis_name="s",
        num_cores=_NCORES, num_subcores=_NSUB)


def _sc_degree(ei, ones, zeros16):
    @functools.partial(
        pl.kernel, mesh=_mesh(),
        compiler_params=pltpu.CompilerParams(use_tc_tiling_on_sc=False),
        out_type=jax.ShapeDtypeStruct((_NCORES, _NPAD, _DEGW), jnp.float32),
        scratch_types=[
            pltpu.VMEM((_EPW,), jnp.int32),
            pltpu.VMEM((_SCH, _DEGW), jnp.float32),
            pltpu.VMEM_SHARED((_NPAD, _DEGW), jnp.float32),
            [pltpu.SemaphoreType.DMA] * 4,
        ])
    def deg_kernel(ei_hbm, ones_hbm, z_hbm, out_hbm, didx, ones_v, acc, sem):
        cid = lax.axis_index("c")
        sid = lax.axis_index("s")
        wid = sid * _NCORES + cid
        pltpu.sync_copy(z_hbm, acc.at[pl.ds(sid * _RPT, _RPT)])
        pltpu.sync_copy(ones_hbm, ones_v)
        pltpu.sync_copy(ei_hbm.at[1, pl.ds(wid * _EPW, _EPW)], didx)
        plsc.subcore_barrier()

        def dst_at(c):
            return acc.at[didx.at[pl.ds(c * _SCH, _SCH)]]

        for k in range(4):
            pltpu.async_copy(ones_v, dst_at(k), sem[k], add=True)

        def chunk(j, c):
            for k in range(4):
                pltpu.make_async_copy(ones_v, dst_at(4 * (j - 1) + k),
                                      sem[k]).wait()
                pltpu.async_copy(ones_v, dst_at(4 * j + k), sem[k],
                                 add=True)
            return c

        lax.fori_loop(1, _SNCH // 4, chunk, None)
        c0 = 4 * (_SNCH // 4 - 1)
        tb = _SNCH * _SCH
        tail = pltpu.make_async_copy(
            ones_v.at[pl.ds(0, _STAIL)],
            acc.at[didx.at[pl.ds(tb, _STAIL)]], sem[3])
        for k in range(4):
            pltpu.make_async_copy(ones_v, dst_at(c0 + k), sem[k]).wait()
        for k in range(_SNCH - c0 - 4):
            pltpu.async_copy(ones_v, dst_at(c0 + 4 + k), sem[k], add=True)
        pltpu.async_copy(ones_v.at[pl.ds(0, _STAIL)],
                         acc.at[didx.at[pl.ds(tb, _STAIL)]], sem[3],
                         add=True)
        for k in range(_SNCH - c0 - 4):
            pltpu.make_async_copy(ones_v, dst_at(c0 + 4 + k), sem[k]).wait()
        tail.wait()
        plsc.subcore_barrier()
        pltpu.sync_copy(acc.at[pl.ds(sid * _RPT, _RPT)],
                        out_hbm.at[cid, pl.ds(sid * _RPT, _RPT)])

    return deg_kernel(ei, ones, zeros16)


def _sc_scatter(hh, ei, zeros16):
    @functools.partial(
        pl.kernel, mesh=_mesh(),
        compiler_params=pltpu.CompilerParams(use_tc_tiling_on_sc=False),
        out_type=jax.ShapeDtypeStruct((_NCORES, _NPAD, _DHID), jnp.float32),
        scratch_types=[
            pltpu.VMEM((_EPW,), jnp.int32),
            pltpu.VMEM((_EPW,), jnp.int32),
            pltpu.VMEM((4, _SCH, _DHID), jnp.float32),
            pltpu.VMEM((_STAIL, _DHID), jnp.float32),
            pltpu.VMEM_SHARED((_NPAD, _DHID), jnp.float32),
            [pltpu.SemaphoreType.DMA] * 4,
            [pltpu.SemaphoreType.DMA] * 4,
        ])
    def scat_kernel(hh_hbm, ei_hbm, z_hbm, out_hbm,
                    sidx, didx, rows, rowt, acc, gsem, ssem):
        cid = lax.axis_index("c")
        sid = lax.axis_index("s")
        wid = sid * _NCORES + cid
        pltpu.sync_copy(z_hbm, acc.at[pl.ds(sid * _RPT, _RPT)])
        pltpu.sync_copy(ei_hbm.at[0, pl.ds(wid * _EPW, _EPW)], sidx)
        pltpu.sync_copy(ei_hbm.at[1, pl.ds(wid * _EPW, _EPW)], didx)
        plsc.subcore_barrier()

        def gat(c, k):
            return pltpu.make_async_copy(
                hh_hbm.at[sidx.at[pl.ds(c * _SCH, _SCH)]], rows.at[k],
                gsem[k])

        def sca(c, k):
            return pltpu.make_async_copy(
                rows.at[k], acc.at[didx.at[pl.ds(c * _SCH, _SCH)]], ssem[k])

        def start_gat(c, k):
            pltpu.async_copy(hh_hbm.at[sidx.at[pl.ds(c * _SCH, _SCH)]],
                             rows.at[k], gsem[k])

        def start_sca(c, k):
            pltpu.async_copy(rows.at[k],
                             acc.at[didx.at[pl.ds(c * _SCH, _SCH)]],
                             ssem[k], add=True)

        for k in range(4):
            start_gat(k, k)

        def quad(j, carry):
            c0 = 4 * j
            for k in range(4):
                gat(c0 + k, k).wait()
                start_sca(c0 + k, k)
            for k in range(4):
                sca(c0 + k, k).wait()
                start_gat(c0 + 4 + k, k)
            return carry

        nq = _SNCH // 4 - 1
        lax.fori_loop(0, nq, quad, None)
        c0 = 4 * nq
        tb = _SNCH * _SCH
        for k in range(4):
            gat(c0 + k, k).wait()
            start_sca(c0 + k, k)
        for k in range(_SNCH - c0 - 4):
            sca(c0 + k, k).wait()
            start_gat(c0 + 4 + k, k)
        pltpu.async_copy(hh_hbm.at[sidx.at[pl.ds(tb, _STAIL)]], rowt,
                         gsem[3])
        for k in range(_SNCH - c0 - 4, 4):
            sca(c0 + k, k).wait()
        for k in range(_SNCH - c0 - 4):
            gat(c0 + 4 + k, k).wait()
            start_sca(c0 + 4 + k, k)
        pltpu.make_async_copy(hh_hbm.at[sidx.at[pl.ds(tb, _STAIL)]], rowt,
                              gsem[3]).wait()
        pltpu.sync_copy(rowt, acc.at[didx.at[pl.ds(tb, _STAIL)]], add=True)
        for k in range(_SNCH - c0 - 4):
            sca(c0 + 4 + k, k).wait()
        plsc.subcore_barrier()
        pltpu.sync_copy(acc.at[pl.ds(sid * _RPT, _RPT)],
                        out_hbm.at[cid, pl.ds(sid * _RPT, _RPT)])

    return scat_kernel(hh, ei, zeros16)


def _dinv_packed(deg_ref):
    dsum = deg_ref[0] + deg_ref[1] + 1.0
    return lax.rsqrt(jnp.maximum(dsum, 1.0))


def _tc1_body(xp_ref, w1_ref, deg_ref, o_ref):
    hp = jnp.dot(xp_ref[...], w1_ref[...],
                 preferred_element_type=jnp.float32)
    o_ref[...] = hp * _dinv_packed(deg_ref)


def _tc1(xp, W1blk, degp):
    return pl.pallas_call(
        _tc1_body,
        grid=(_GRID,),
        in_specs=[
            pl.BlockSpec((_RBP, _PK * _DIN), lambda i: (i, 0)),
            pl.BlockSpec((_PK * _DIN, 128), lambda i: (0, 0)),
            pl.BlockSpec((_NCORES, _RBP, 128), lambda i: (0, i, 0)),
        ],
        out_specs=pl.BlockSpec((_RBP, 128), lambda i: (i, 0)),
        out_shape=jax.ShapeDtypeStruct((_PROWS, 128), jnp.float32),
    )(xp, W1blk, degp)


def _tc2_body(acc_ref, hh_ref, deg_ref, w2_ref, b1_ref, o_ref):
    dinv = _dinv_packed(deg_ref)
    s = acc_ref[0] + acc_ref[1] + hh_ref[...]
    h1 = jnp.maximum(s * dinv + b1_ref[...], 0.0)
    h2 = jnp.dot(h1, w2_ref[...], preferred_element_type=jnp.float32)
    o_ref[...] = h2 * dinv


def _tc2(acc1p, hhp, degp, W2blk, b1p):
    return pl.pallas_call(
        _tc2_body,
        grid=(_GRID,),
        in_specs=[
            pl.BlockSpec((_NCORES, _RBP, 128), lambda i: (0, i, 0)),
            pl.BlockSpec((_RBP, 128), lambda i: (i, 0)),
            pl.BlockSpec((_NCORES, _RBP, 128), lambda i: (0, i, 0)),
            pl.BlockSpec((128, 128), lambda i: (0, 0)),
            pl.BlockSpec((1, 128), lambda i: (0, 0)),
        ],
        out_specs=pl.BlockSpec((_RBP, 128), lambda i: (i, 0)),
        out_shape=jax.ShapeDtypeStruct((_PROWS, 128), jnp.float32),
    )(acc1p, hhp, degp, W2blk, b1p)


def _tc3_body(acc_ref, hh_ref, deg_ref, b2_ref, o_ref):
    dinv = _dinv_packed(deg_ref)
    s = acc_ref[0] + acc_ref[1] + hh_ref[...]
    zp = s * dinv + b2_ref[...]
    col = lax.broadcasted_iota(jnp.int32, (_RBP, _DHID), 1)
    mask = col < _NCLS
    mparts = []
    for a in range(_PK):
        z = zp[:, a * _DHID:(a + 1) * _DHID]
        neg = jnp.full_like(z, -3.0e38)
        m = jnp.max(jnp.where(mask, z, neg), axis=1, keepdims=True)
        mparts.append(jnp.broadcast_to(m, (_RBP, _DHID)))
    mb = jnp.concatenate(mparts, axis=1)
    lane = lax.broadcasted_iota(jnp.int32, (_RBP, 128), 1)
    maskp = lax.rem(lane, _DHID) < _NCLS
    e = jnp.where(maskp, jnp.exp(zp - mb), 0.0)
    gi = lax.broadcasted_iota(jnp.int32, (128, 128), 0) // _DHID
    gj = lax.broadcasted_iota(jnp.int32, (128, 128), 1) // _DHID
    gmat = (gi == gj).astype(jnp.float32)
    gsum = jnp.dot(e, gmat, preferred_element_type=jnp.float32,
                   precision=lax.Precision.HIGHEST)
    o_ref[...] = zp - mb - jnp.log(gsum)


def _tc3(acc2p, hh2p, degp, b2p):
    return pl.pallas_call(
        _tc3_body,
        grid=(_GRID,),
        in_specs=[
            pl.BlockSpec((_NCORES, _RBP, 128), lambda i: (0, i, 0)),
            pl.BlockSpec((_RBP, 128), lambda i: (i, 0)),
            pl.BlockSpec((_NCORES, _RBP, 128), lambda i: (0, i, 0)),
            pl.BlockSpec((1, 128), lambda i: (0, 0)),
        ],
        out_specs=pl.BlockSpec((_RBP, 128), lambda i: (i, 0)),
        out_shape=jax.ShapeDtypeStruct((_PROWS, 128), jnp.float32),
    )(acc2p, hh2p, degp, b2p)


def kernel(x, edge_index, W1, b1, W2, b2):
    ei = edge_index.astype(jnp.int32)
    W2p = jnp.pad(W2, ((0, 0), (0, _DHID - _NCLS)))
    W2blk = jnp.kron(jnp.eye(_PK, dtype=jnp.float32), W2p)
    W1blk = jnp.kron(jnp.eye(_PK, dtype=jnp.float32), W1)
    xp = jnp.pad(x.reshape(_N * _DIN), (0, (_NPAD - _N) * _DIN)).reshape(
        _PROWS, _PK * _DIN)
    b1p = jnp.tile(b1, _PK).reshape(1, 128)
    b2p = jnp.tile(jnp.pad(b2, (0, _DHID - _NCLS)), _PK).reshape(1, 128)
    ones = jnp.ones((_SCH, _DEGW), jnp.float32)
    zeros16 = jnp.zeros((_RPT, _DHID), jnp.float32)

    deg_parts = _sc_degree(ei, ones, zeros16)
    degp = deg_parts.reshape(_NCORES, _PROWS, 128)
    hhp = _tc1(xp, W1blk, degp)
    acc1 = _sc_scatter(hhp.reshape(_NPAD, _DHID), ei, zeros16)
    hh2p = _tc2(acc1.reshape(_NCORES, _PROWS, 128), hhp, degp, W2blk, b1p)
    acc2 = _sc_scatter(hh2p.reshape(_NPAD, _DHID), ei, zeros16)
    outp = _tc3(acc2.reshape(_NCORES, _PROWS, 128), hh2p, degp, b2p)
    return outp.reshape(_NPAD, _DHID)[:_N, :_NCLS]
